# Initial kernel scaffold; baseline (speedup 1.0000x reference)
import functools

import jax
import jax.numpy as jnp
import numpy as np
from jax import lax
from jax.experimental import pallas as pl
from jax.experimental.pallas import tpu as pltpu
from jax.experimental.pallas import tpu_sc as plsc

BUFF_SIZE = 1000000
D = 64
N_SAMP = 16384
KEEP = BUFF_SIZE - N_SAMP

_PERM_IDX = np.asarray(
    jax.jit(lambda: jax.random.permutation(jax.random.key(1), BUFF_SIZE)[:N_SAMP])()
).astype(np.int32)

_info = plsc.get_sparse_core_info()
_NC, _NS = _info.num_cores, _info.num_subcores
NW = _NC * _NS
B_PER_W = N_SAMP // NW
CHUNK = 128
CHUNKS = B_PER_W // CHUNK

_IDX_ARR = _PERM_IDX.reshape(NW, CHUNKS, CHUNK)


def _fifo_copy_body(samples_ref, buff_ref, out_ref, sem0, sem1):
    head = pltpu.make_async_copy(samples_ref, out_ref.at[pl.ds(0, N_SAMP)], sem0)
    tail = pltpu.make_async_copy(
        buff_ref.at[pl.ds(0, KEEP)], out_ref.at[pl.ds(N_SAMP, KEEP)], sem1
    )
    head.start()
    tail.start()
    head.wait()
    tail.wait()


_fifo_copy = pl.pallas_call(
    _fifo_copy_body,
    out_shape=jax.ShapeDtypeStruct((BUFF_SIZE, D), jnp.float32),
    in_specs=[
        pl.BlockSpec(memory_space=pltpu.ANY),
        pl.BlockSpec(memory_space=pltpu.ANY),
    ],
    out_specs=pl.BlockSpec(memory_space=pltpu.ANY),
    scratch_shapes=[pltpu.SemaphoreType.DMA, pltpu.SemaphoreType.DMA],
)


@functools.partial(
    pl.kernel,
    mesh=plsc.VectorSubcoreMesh(core_axis_name="c", subcore_axis_name="s"),
    out_type=jax.ShapeDtypeStruct((N_SAMP, D), jnp.float32),
    scratch_types=[
        pltpu.VMEM((CHUNKS, CHUNK), jnp.int32),
        pltpu.VMEM((B_PER_W, D), jnp.float32),
        pltpu.SemaphoreType.DMA,
    ],
)
def _gather_kernel(table_hbm, idx_hbm, out_hbm, idx_v, rows_v, sem):
    wid = lax.axis_index("s") * _NC + lax.axis_index("c")
    pltpu.sync_copy(idx_hbm.at[wid], idx_v)
    copies = [
        pltpu.async_copy(
            table_hbm.at[idx_v.at[i]],
            rows_v.at[pl.ds(i * CHUNK, CHUNK)],
            sem,
        )
        for i in range(CHUNKS)
    ]
    for c in copies:
        c.wait()
    pltpu.sync_copy(rows_v, out_hbm.at[pl.ds(wid * B_PER_W, B_PER_W)])


def kernel(buff, samples):
    new_buff = _fifo_copy(samples, buff)
    sampled = _gather_kernel(new_buff, jnp.asarray(_IDX_ARR))
    return new_buff, sampled


# speedup vs baseline: 9.5585x; 9.5585x over previous
import functools

import jax
import jax.numpy as jnp
import numpy as np
from jax import lax
from jax.experimental import pallas as pl
from jax.experimental.pallas import tpu as pltpu
from jax.experimental.pallas import tpu_sc as plsc

BUFF_SIZE = 1000000
D = 64
N_SAMP = 16384
KEEP = BUFF_SIZE - N_SAMP

def _compute_perm_idx():
    with jax.set_mesh(None):
        cpu = jax.local_devices(backend="cpu")[0]
        with jax.default_device(cpu):
            p = jax.random.permutation(jax.random.key(1), BUFF_SIZE)
            return np.asarray(p[:N_SAMP]).astype(np.int32)


_PERM_IDX = _compute_perm_idx()

_NC, _NS = 2, 16
NW = _NC * _NS
B_PER_W = N_SAMP // NW
CHUNK = 128
CHUNKS = B_PER_W // CHUNK

def _remap_to_table(idx):
    j_blk = idx // 16384
    t = idx % 16384
    return j_blk * 16384 + 2 * (t % 8192) + t // 8192


_BLK = N_SAMP
_N_BULK = (KEEP + _BLK - 1) // _BLK


_TBL_BLK = _BLK // 2
_N_TBL_BLKS = _N_BULK + 1
_TBL_ROWS = _N_TBL_BLKS * _TBL_BLK


def _pack_table(x, tbl_blk):
    xt = jnp.swapaxes(x, 0, 1)
    tbl_blk[:, :D] = xt[:_TBL_BLK]
    tbl_blk[:, D:] = xt[_TBL_BLK:]


def _copy_block_body(src_blk, out_blk, tbl_blk):
    x = src_blk[...]
    out_blk[...] = x
    _pack_table(x, tbl_blk)


_bulk_copy = pl.pallas_call(
    _copy_block_body,
    grid=(_N_BULK,),
    out_shape=(
        jax.ShapeDtypeStruct((D, BUFF_SIZE), jnp.float32),
        jax.ShapeDtypeStruct((_TBL_ROWS, 2 * D), jnp.float32),
    ),
    in_specs=[pl.BlockSpec((D, _BLK), lambda j: (0, j))],
    out_specs=(
        pl.BlockSpec((D, _BLK), lambda j: (0, j + 1)),
        pl.BlockSpec((_TBL_BLK, 2 * D), lambda j: (j + 1, 0)),
    ),
)


def _head_body(_prev_ref, _prev_tbl_ref, samples_blk, out_blk, tbl_blk):
    x = samples_blk[...]
    out_blk[...] = x
    _pack_table(x, tbl_blk)


_head_copy = pl.pallas_call(
    _head_body,
    grid=(1,),
    out_shape=(
        jax.ShapeDtypeStruct((D, BUFF_SIZE), jnp.float32),
        jax.ShapeDtypeStruct((_TBL_ROWS, 2 * D), jnp.float32),
    ),
    in_specs=[
        pl.BlockSpec(memory_space=pl.ANY),
        pl.BlockSpec(memory_space=pl.ANY),
        pl.BlockSpec((D, N_SAMP), lambda i: (0, 0)),
    ],
    out_specs=(
        pl.BlockSpec((D, N_SAMP), lambda i: (0, 0)),
        pl.BlockSpec((_TBL_BLK, 2 * D), lambda i: (0, 0)),
    ),
    input_output_aliases={0: 0, 1: 1},
)


_TBL_FLAT_ROWS = 2 * _TBL_ROWS
_GIDX_ARR = _remap_to_table(_PERM_IDX).astype(np.int32).reshape(NW, CHUNKS, CHUNK)


@functools.lru_cache(maxsize=None)
def _make_gather():
    mesh = plsc.VectorSubcoreMesh(core_axis_name="c", subcore_axis_name="s")

    @functools.partial(
        pl.kernel,
        mesh=mesh,
        compiler_params=pltpu.CompilerParams(use_tc_tiling_on_sc=False),
        out_type=jax.ShapeDtypeStruct((N_SAMP, D), jnp.float32),
        scratch_types=[
            pltpu.VMEM((CHUNKS, CHUNK), jnp.int32),
            pltpu.VMEM((B_PER_W, D), jnp.float32),
            pltpu.SemaphoreType.DMA,
        ],
    )
    def _gather_kernel(table_hbm, idx_hbm, out_hbm, idx_v, rows_v, sem):
        wid = lax.axis_index("s") * _NC + lax.axis_index("c")
        pltpu.sync_copy(idx_hbm.at[wid], idx_v)
        copies = [
            pltpu.async_copy(
                table_hbm.at[idx_v.at[i]],
                rows_v.at[pl.ds(i * CHUNK, CHUNK)],
                sem,
            )
            for i in range(CHUNKS)
        ]
        for c in copies:
            c.wait()
        pltpu.sync_copy(rows_v, out_hbm.at[pl.ds(wid * B_PER_W, B_PER_W)])

    return _gather_kernel


def kernel(buff, samples):
    out_t, tbl = _bulk_copy(buff.T)
    out_t, tbl = _head_copy(out_t, tbl, samples.T)
    new_buff = out_t.T
    table = tbl.reshape(_TBL_FLAT_ROWS, D)
    sampled = _make_gather()(table, jnp.asarray(_GIDX_ARR))
    return new_buff, sampled


# speedup vs baseline: 9.6321x; 1.0077x over previous
import functools

import jax
import jax.numpy as jnp
import numpy as np
from jax import lax
from jax.experimental import pallas as pl
from jax.experimental.pallas import tpu as pltpu
from jax.experimental.pallas import tpu_sc as plsc

BUFF_SIZE = 1000000
D = 64
N_SAMP = 16384
KEEP = BUFF_SIZE - N_SAMP

def _compute_perm_idx():
    with jax.set_mesh(None):
        cpu = jax.local_devices(backend="cpu")[0]
        with jax.default_device(cpu):
            p = jax.random.permutation(jax.random.key(1), BUFF_SIZE)
            return np.asarray(p[:N_SAMP]).astype(np.int32)


_PERM_IDX = _compute_perm_idx()

_NC, _NS = 2, 16
NW = _NC * _NS
B_PER_W = N_SAMP // NW
CHUNK = 128
CHUNKS = B_PER_W // CHUNK

def _remap_to_table(idx):
    j_blk = idx // 16384
    t = idx % 16384
    return j_blk * 16384 + 2 * (t % 8192) + t // 8192


_BLK = N_SAMP
_N_BULK = (KEEP + _BLK - 1) // _BLK


_TBL_BLK = _BLK // 2
_N_TBL_BLKS = _N_BULK + 1
_TBL_ROWS = _N_TBL_BLKS * _TBL_BLK


def _pack_table(x, tbl_blk):
    xt = jnp.swapaxes(x, 0, 1)
    tbl_blk[:, :D] = xt[:_TBL_BLK]
    tbl_blk[:, D:] = xt[_TBL_BLK:]


def _copy_block_body(samples_blk, buff_blk, out_blk, tbl_blk):
    j = pl.program_id(0)

    @pl.when(j == 0)
    def _():
        x = samples_blk[...]
        out_blk[...] = x
        _pack_table(x, tbl_blk)

    @pl.when(j > 0)
    def _():
        x = buff_blk[...]
        out_blk[...] = x
        _pack_table(x, tbl_blk)


_fifo_copy = pl.pallas_call(
    _copy_block_body,
    grid=(_N_TBL_BLKS,),
    out_shape=(
        jax.ShapeDtypeStruct((D, BUFF_SIZE), jnp.float32),
        jax.ShapeDtypeStruct((_TBL_ROWS, 2 * D), jnp.float32),
    ),
    in_specs=[
        pl.BlockSpec((D, _BLK), lambda j: (0, 0)),
        pl.BlockSpec((D, _BLK), lambda j: (0, jnp.maximum(j - 1, 0))),
    ],
    out_specs=(
        pl.BlockSpec((D, _BLK), lambda j: (0, j)),
        pl.BlockSpec((_TBL_BLK, 2 * D), lambda j: (j, 0)),
    ),
)


_TBL_FLAT_ROWS = 2 * _TBL_ROWS
_GIDX_ARR = _remap_to_table(_PERM_IDX).astype(np.int32).reshape(NW, CHUNKS, CHUNK)


@functools.lru_cache(maxsize=None)
def _make_gather():
    mesh = plsc.VectorSubcoreMesh(core_axis_name="c", subcore_axis_name="s")

    @functools.partial(
        pl.kernel,
        mesh=mesh,
        compiler_params=pltpu.CompilerParams(use_tc_tiling_on_sc=False),
        out_type=jax.ShapeDtypeStruct((N_SAMP, D), jnp.float32),
        scratch_types=[
            pltpu.VMEM((CHUNKS, CHUNK), jnp.int32),
            pltpu.VMEM((B_PER_W, D), jnp.float32),
            pltpu.SemaphoreType.DMA,
        ],
    )
    def _gather_kernel(table_hbm, idx_hbm, out_hbm, idx_v, rows_v, sem):
        wid = lax.axis_index("s") * _NC + lax.axis_index("c")
        pltpu.sync_copy(idx_hbm.at[wid], idx_v)
        copies = [
            pltpu.async_copy(
                table_hbm.at[idx_v.at[i]],
                rows_v.at[pl.ds(i * CHUNK, CHUNK)],
                sem,
            )
            for i in range(CHUNKS)
        ]
        for c in copies:
            c.wait()
        pltpu.sync_copy(rows_v, out_hbm.at[pl.ds(wid * B_PER_W, B_PER_W)])

    return _gather_kernel


def kernel(buff, samples):
    out_t, tbl = _fifo_copy(samples.T, buff.T)
    new_buff = out_t.T
    table = tbl.reshape(_TBL_FLAT_ROWS, D)
    sampled = _make_gather()(table, jnp.asarray(_GIDX_ARR))
    return new_buff, sampled
